# baseline (speedup 1.0000x reference)
import functools

import jax
import jax.numpy as jnp
from jax import lax
from jax.experimental import pallas as pl
from jax.experimental.pallas import tpu as pltpu
from jax.experimental.pallas import tpu_sc as plsc

NC, NS, L = 2, 16, 16
NW = NC * NS
B, N, S = 64, 128, 2
HW = 128 * 128
BPW = B // NW
NCHUNK = N // L


def _sc_gather(out_flat, ind_flat):
    mesh = plsc.VectorSubcoreMesh(
        core_axis_name="c", subcore_axis_name="s",
        num_cores=NC, num_subcores=NS)

    @functools.partial(
        pl.kernel,
        out_type=jax.ShapeDtypeStruct((S, B * N), jnp.float32),
        mesh=mesh,
        scratch_types=[
            pltpu.VMEM((BPW * N,), jnp.int32),
            pltpu.VMEM((N,), jnp.int32),
            pltpu.VMEM((N,), jnp.int32),
            pltpu.VMEM((N,), jnp.int32),
            pltpu.VMEM((N,), jnp.int32),
            pltpu.VMEM((BPW * N,), jnp.float32),
            pltpu.VMEM((BPW * N,), jnp.float32),
            pltpu.SemaphoreType.DMA,
        ],
    )
    def k(out_hbm, ind_hbm, pred_hbm,
          ind_v, idx00, idx01, idx10, idx11, p0_v, p1_v, sem):
        wid = lax.axis_index("s") * NC + lax.axis_index("c")
        b0 = wid * BPW
        pltpu.sync_copy(ind_hbm.at[pl.ds(b0 * N, BPW * N)], ind_v)
        idx = ((idx00, idx01), (idx10, idx11))
        for j in range(BPW):
            base = (b0 + j) * (S * HW)
            for i in range(NCHUNK):
                c = ind_v[pl.ds(j * N + i * L, L)]
                idx[j][0][pl.ds(i * L, L)] = c + base
                idx[j][1][pl.ds(i * L, L)] = c + (base + HW)
        cps = []
        for j in range(BPW):
            cps.append(pltpu.async_copy(
                out_hbm.at[idx[j][0]], p0_v.at[pl.ds(j * N, N)], sem))
            cps.append(pltpu.async_copy(
                out_hbm.at[idx[j][1]], p1_v.at[pl.ds(j * N, N)], sem))
        for cp in cps:
            cp.wait()
        w0 = pltpu.async_copy(p0_v, pred_hbm.at[0, pl.ds(b0 * N, BPW * N)], sem)
        w1 = pltpu.async_copy(p1_v, pred_hbm.at[1, pl.ds(b0 * N, BPW * N)], sem)
        w0.wait()
        w1.wait()

    return k(out_flat, ind_flat)


def _tc_loss(pred, target_flat, mask):
    def k(p_ref, t_ref, m_ref, o_ref):
        p0 = p_ref[0]
        p1 = p_ref[1]
        t = t_ref[...].reshape(B, N, S)
        t0 = t[:, :, 0]
        t1 = t[:, :, 1]
        m = m_ref[...]
        num = jnp.sum(jnp.abs(p0 * m - t0 * m) + jnp.abs(p1 * m - t1 * m),
                      keepdims=True)
        den = jnp.sum(m, keepdims=True) + 0.0001
        o_ref[...] = num / den

    r = pl.pallas_call(
        k, out_shape=jax.ShapeDtypeStruct((1, 1), jnp.float32),
    )(pred, target_flat, mask)
    return r[0, 0]


def kernel(out, target, ind, mask):
    pred = _sc_gather(out.reshape(-1), ind.reshape(-1))
    return _tc_loss(pred.reshape(S, B, N), target.reshape(B, S * N), mask)


# speedup vs baseline: 1.1037x; 1.1037x over previous
import functools

import jax
import jax.numpy as jnp
from jax import lax
from jax.experimental import pallas as pl
from jax.experimental.pallas import tpu as pltpu
from jax.experimental.pallas import tpu_sc as plsc

NC, NS, L = 1, 16, 16
NW = NC * NS
B, N, S = 64, 128, 2
HW = 128 * 128
BPW = B // NW
PW = BPW * N
NCHUNK = PW // L


def _sc_loss(out_flat, ind_flat, mask_flat, target_2d):
    mesh = plsc.VectorSubcoreMesh(
        core_axis_name="c", subcore_axis_name="s",
        num_cores=NC, num_subcores=NS)

    @functools.partial(
        pl.kernel,
        out_type=jax.ShapeDtypeStruct((L,), jnp.float32),
        mesh=mesh,
        scratch_types=[
            pltpu.VMEM((PW,), jnp.int32),
            pltpu.VMEM((PW,), jnp.float32),
            pltpu.VMEM((PW,), jnp.float32),
            pltpu.VMEM((PW,), jnp.float32),
            pltpu.VMEM((PW,), jnp.int32),
            pltpu.VMEM((PW,), jnp.int32),
            pltpu.VMEM((PW,), jnp.int32),
            pltpu.VMEM((PW,), jnp.int32),
            pltpu.VMEM((PW,), jnp.float32),
            pltpu.VMEM((PW,), jnp.float32),
            pltpu.VMEM((2 * L,), jnp.float32),
            pltpu.VMEM((NW, 2 * L), jnp.float32),
            pltpu.VMEM_SHARED((NW, 2 * L), jnp.float32),
            pltpu.VMEM((L,), jnp.float32),
            pltpu.SemaphoreType.DMA,
        ],
    )
    def k(out_hbm, ind_hbm, mask_hbm, tgt_hbm, o_hbm,
          ind_v, m_v, t0_v, t1_v, idx0_v, idx1_v, tix0_v, tix1_v, p0_v, p1_v,
          part_v, all_v, shared, res_v, sem):
        wid = lax.axis_index("s") * NC + lax.axis_index("c")
        p0 = wid * PW
        iota = lax.iota(jnp.int32, L)
        c0 = pltpu.async_copy(ind_hbm.at[pl.ds(p0, PW)], ind_v, sem)
        c1 = pltpu.async_copy(mask_hbm.at[pl.ds(p0, PW)], m_v, sem)
        for i in range(NCHUNK):
            sl = pl.ds(i * L, L)
            e = (p0 + i * L + iota) * S
            tix0_v[sl] = e
            tix1_v[sl] = e + 1
        gs = []
        for j in range(BPW):
            gs.append(pltpu.async_copy(
                tgt_hbm.at[tix0_v.at[pl.ds(j * N, N)]],
                t0_v.at[pl.ds(j * N, N)], sem))
            gs.append(pltpu.async_copy(
                tgt_hbm.at[tix1_v.at[pl.ds(j * N, N)]],
                t1_v.at[pl.ds(j * N, N)], sem))
        c0.wait()
        for j in range(BPW):
            base = (wid * BPW + j) * (S * HW)
            for i in range(N // L):
                c = ind_v[pl.ds(j * N + i * L, L)]
                idx0_v[pl.ds(j * N + i * L, L)] = c + base
                idx1_v[pl.ds(j * N + i * L, L)] = c + (base + HW)
        for j in range(BPW):
            gs.append(pltpu.async_copy(
                out_hbm.at[idx0_v.at[pl.ds(j * N, N)]],
                p0_v.at[pl.ds(j * N, N)], sem))
            gs.append(pltpu.async_copy(
                out_hbm.at[idx1_v.at[pl.ds(j * N, N)]],
                p1_v.at[pl.ds(j * N, N)], sem))
        c1.wait()
        for g in gs:
            g.wait()
        lacc = jnp.zeros((L,), jnp.float32)
        macc = jnp.zeros((L,), jnp.float32)
        for i in range(NCHUNK):
            sl = pl.ds(i * L, L)
            m = m_v[sl]
            lacc = (lacc + jnp.abs(p0_v[sl] * m - t0_v[sl] * m)
                    + jnp.abs(p1_v[sl] * m - t1_v[sl] * m))
            macc = macc + m
        part_v[pl.ds(0, L)] = lacc
        part_v[pl.ds(L, L)] = macc
        pltpu.sync_copy(part_v, shared.at[wid])
        plsc.subcore_barrier()

        @pl.when(wid == 0)
        def _():
            pltpu.sync_copy(shared, all_v)
            lt = jnp.zeros((L,), jnp.float32)
            mt = jnp.zeros((L,), jnp.float32)
            for w in range(NW):
                lt = lt + all_v[w, pl.ds(0, L)]
                mt = mt + all_v[w, pl.ds(L, L)]
            num = jnp.float32(0.0)
            den = jnp.float32(0.0001)
            for i in range(L):
                num = num + lt[i]
                den = den + mt[i]
            res_v[...] = (jnp.full((L,), num, jnp.float32)
                          / jnp.full((L,), den, jnp.float32))
            pltpu.sync_copy(res_v, o_hbm)

    return k(out_flat, ind_flat, mask_flat, target_2d)


def kernel(out, target, ind, mask):
    res = _sc_loss(out.reshape(-1), ind.reshape(-1), mask.reshape(-1),
                   target.reshape(-1))
    return res[0]
